# Initial kernel scaffold; baseline (speedup 1.0000x reference)
import jax
import jax.numpy as jnp
from jax.experimental import pallas as pl


def kernel(x, edge_index, W_self1, W_neigh1, b1, W_self2, W_neigh2, b2):
    raise NotImplementedError("write your pallas kernel here")


# speedup vs baseline: 2.9216x; 2.9216x over previous
import functools

import jax
import jax.numpy as jnp
from jax import lax
from jax.experimental import pallas as pl
from jax.experimental.pallas import tpu as pltpu
from jax.experimental.pallas import tpu_sc as plsc

N = 10000
E = 320000
D = 128
DEXT = 144

NC = 2
NS = 16
NW = NC * NS

CHUNK = 128
GRP = 8
KCH = 80
EPT = KCH * CHUNK
EPAD = EPT * NW

NPAD = 10240
RPT = NPAD // NS
DUMMY = N

BN = 1024


def _sc_aggregate(feat, src_t, dst_t, z_feat, fw):
    scratch = dict(
        src_v=pltpu.VMEM((GRP, CHUNK), jnp.int32),
        dst_v=pltpu.VMEM((GRP, CHUNK), jnp.int32),
        rows_v=pltpu.VMEM((CHUNK, fw), jnp.float32),
        acc_sh=pltpu.VMEM_SHARED((NPAD, fw), jnp.float32),
        sem=pltpu.SemaphoreType.DMA,
    )

    mesh = plsc.VectorSubcoreMesh(core_axis_name="c", subcore_axis_name="s")

    @functools.partial(
        pl.kernel,
        out_type=jax.ShapeDtypeStruct((NC, NPAD, fw), jnp.float32),
        mesh=mesh, scratch_types=scratch,
        compiler_params=pltpu.CompilerParams(use_tc_tiling_on_sc=False))
    def run(feat_hbm, src_hbm, dst_hbm, zf_hbm, acc_out, *,
            src_v, dst_v, rows_v, acc_sh, sem):
        c = lax.axis_index("c")
        s = lax.axis_index("s")
        wid = s * NC + c
        base = s * RPT

        pltpu.sync_copy(zf_hbm, acc_sh.at[pl.ds(base, RPT)])
        plsc.subcore_barrier()

        def body(g, carry):
            off = wid * KCH + g * GRP
            pltpu.sync_copy(src_hbm.at[pl.ds(off, GRP)], src_v)
            pltpu.sync_copy(dst_hbm.at[pl.ds(off, GRP)], dst_v)
            for j in range(GRP):
                pltpu.async_copy(feat_hbm.at[src_v.at[j]], rows_v, sem).wait()
                pltpu.sync_copy(rows_v, acc_sh.at[dst_v.at[j]], add=True)
            return carry

        lax.fori_loop(0, KCH // GRP, body, 0)
        plsc.subcore_barrier()

        pltpu.sync_copy(acc_sh.at[pl.ds(base, RPT)],
                        acc_out.at[c].at[pl.ds(base, RPT)])

    return run(feat, src_t, dst_t, z_feat)


def _tc_layer1_kernel(x_ref, a0_ref, a1_ref, ws_ref, wn_ref, b_ref, out_ref):
    acc = a0_ref[...] + a1_ref[...]
    deg = acc[:, D:D + 1]
    inv = 1.0 / jnp.maximum(deg, 1.0)
    agg = acc[:, :D] * inv
    out = (jnp.dot(x_ref[...], ws_ref[...], preferred_element_type=jnp.float32)
           + jnp.dot(agg, wn_ref[...], preferred_element_type=jnp.float32)
           + b_ref[...])
    out_ref[...] = jnp.maximum(out, 0.0)


def _tc_layer2_kernel(h_ref, a0_ref, a1_ref, d0_ref, d1_ref,
                      ws_ref, wn_ref, b_ref, out_ref):
    deg = d0_ref[...] + d1_ref[...]
    inv = 1.0 / jnp.maximum(deg, 1.0)
    agg = (a0_ref[...] + a1_ref[...]) * inv
    out = (jnp.dot(h_ref[...], ws_ref[...], preferred_element_type=jnp.float32)
           + jnp.dot(agg, wn_ref[...], preferred_element_type=jnp.float32)
           + b_ref[...])
    out_ref[...] = jnp.maximum(out, 0.0)


def _row_spec(w):
    return pl.BlockSpec((BN, w), lambda i: (i, 0))


_W_SPEC = pl.BlockSpec((D, D), lambda i: (0, 0))
_B_SPEC = pl.BlockSpec((1, D), lambda i: (0, 0))


def _tc_layer1(x, acc, W_self, W_neigh, b):
    return pl.pallas_call(
        _tc_layer1_kernel,
        grid=(NPAD // BN,),
        in_specs=[_row_spec(D), _row_spec(DEXT), _row_spec(DEXT),
                  _W_SPEC, _W_SPEC, _B_SPEC],
        out_specs=_row_spec(D),
        out_shape=jax.ShapeDtypeStruct((NPAD, D), jnp.float32),
    )(x, acc[0], acc[1], W_self, W_neigh, b.reshape(1, D))


def _tc_layer2(h, acc, deg0, deg1, W_self, W_neigh, b):
    return pl.pallas_call(
        _tc_layer2_kernel,
        grid=(NPAD // BN,),
        in_specs=[_row_spec(D), _row_spec(D), _row_spec(D),
                  _row_spec(1), _row_spec(1), _W_SPEC, _W_SPEC, _B_SPEC],
        out_specs=_row_spec(D),
        out_shape=jax.ShapeDtypeStruct((NPAD, D), jnp.float32),
    )(h, acc[0], acc[1], deg0, deg1, W_self, W_neigh, b.reshape(1, D))


def kernel(x, edge_index, W_self1, W_neigh1, b1, W_self2, W_neigh2, b2):
    src = edge_index[0]
    dst = edge_index[1]
    pad_e = EPAD - E
    src_t = jnp.concatenate(
        [src, jnp.zeros((pad_e,), jnp.int32)]).reshape(NW * KCH, CHUNK)
    dst_t = jnp.concatenate(
        [dst, jnp.full((pad_e,), DUMMY, jnp.int32)]).reshape(NW * KCH, CHUNK)

    xe = jnp.zeros((NPAD, DEXT), jnp.float32)
    xe = xe.at[:N, :D].set(x).at[:N, D].set(1.0)

    acc1 = _sc_aggregate(xe, src_t, dst_t,
                         jnp.zeros((RPT, DEXT), jnp.float32), DEXT)
    h1 = _tc_layer1(xe[:, :D], acc1, W_self1, W_neigh1, b1)

    acc2 = _sc_aggregate(h1, src_t, dst_t,
                         jnp.zeros((RPT, D), jnp.float32), D)
    deg0 = acc1[0, :, D:D + 1]
    deg1 = acc1[1, :, D:D + 1]
    h2 = _tc_layer2(h1, acc2, deg0, deg1, W_self2, W_neigh2, b2)
    return h2[:N]


# speedup vs baseline: 3.2996x; 1.1294x over previous
import functools

import jax
import jax.numpy as jnp
from jax import lax
from jax.experimental import pallas as pl
from jax.experimental.pallas import tpu as pltpu
from jax.experimental.pallas import tpu_sc as plsc

N = 10000
E = 320000
D = 128
DEXT = 144

NC = 2
NS = 16
NW = NC * NS

CHUNK = 128
GRP = 8
KCH = 80
EPT = KCH * CHUNK
EPAD = EPT * NW

NPAD = 10016
RPT = NPAD // NS
DUMMY = N

BN = 2504


def _sc_aggregate(feat, src_t, dst_t, z_feat, fw):
    scratch = dict(
        src_v=pltpu.VMEM((GRP, CHUNK), jnp.int32),
        dst_v=pltpu.VMEM((GRP, CHUNK), jnp.int32),
        rows0_v=pltpu.VMEM((CHUNK, fw), jnp.float32),
        rows1_v=pltpu.VMEM((CHUNK, fw), jnp.float32),
        acc_sh=pltpu.VMEM_SHARED((NPAD, fw), jnp.float32),
        sem0=pltpu.SemaphoreType.DMA,
        sem1=pltpu.SemaphoreType.DMA,
    )

    mesh = plsc.VectorSubcoreMesh(core_axis_name="c", subcore_axis_name="s")

    @functools.partial(
        pl.kernel,
        out_type=jax.ShapeDtypeStruct((NC, NPAD, fw), jnp.float32),
        mesh=mesh, scratch_types=scratch,
        compiler_params=pltpu.CompilerParams(use_tc_tiling_on_sc=False))
    def run(feat_hbm, src_hbm, dst_hbm, zf_hbm, acc_out, *,
            src_v, dst_v, rows0_v, rows1_v, acc_sh, sem0, sem1):
        c = lax.axis_index("c")
        s = lax.axis_index("s")
        wid = s * NC + c
        base = s * RPT
        bufs = (rows0_v, rows1_v)
        sems = (sem0, sem1)

        pltpu.sync_copy(zf_hbm, acc_sh.at[pl.ds(base, RPT)])
        plsc.subcore_barrier()

        def body(g, carry):
            off = wid * KCH + g * GRP
            pltpu.sync_copy(src_hbm.at[pl.ds(off, GRP)], src_v)
            pltpu.sync_copy(dst_hbm.at[pl.ds(off, GRP)], dst_v)
            cp = pltpu.async_copy(feat_hbm.at[src_v.at[0]], bufs[0], sems[0])
            for j in range(GRP):
                p = j % 2
                if j + 1 < GRP:
                    nxt = pltpu.async_copy(
                        feat_hbm.at[src_v.at[j + 1]], bufs[1 - p], sems[1 - p])
                cp.wait()
                pltpu.sync_copy(bufs[p], acc_sh.at[dst_v.at[j]], add=True)
                if j + 1 < GRP:
                    cp = nxt
            return carry

        lax.fori_loop(0, KCH // GRP, body, 0)
        plsc.subcore_barrier()

        pltpu.sync_copy(acc_sh.at[pl.ds(base, RPT)],
                        acc_out.at[c].at[pl.ds(base, RPT)])

    return run(feat, src_t, dst_t, z_feat)


def _tc_layer1_kernel(x_ref, a0_ref, a1_ref, ws_ref, wn_ref, b_ref, out_ref):
    acc = a0_ref[...] + a1_ref[...]
    deg = acc[:, D:D + 1]
    inv = 1.0 / jnp.maximum(deg, 1.0)
    agg = acc[:, :D] * inv
    out = (jnp.dot(x_ref[...], ws_ref[...], preferred_element_type=jnp.float32)
           + jnp.dot(agg, wn_ref[...], preferred_element_type=jnp.float32)
           + b_ref[...])
    out_ref[...] = jnp.maximum(out, 0.0)


def _tc_layer2_kernel(h_ref, a0_ref, a1_ref, d0_ref, d1_ref,
                      ws_ref, wn_ref, b_ref, out_ref):
    deg = d0_ref[...] + d1_ref[...]
    inv = 1.0 / jnp.maximum(deg, 1.0)
    agg = (a0_ref[...] + a1_ref[...]) * inv
    out = (jnp.dot(h_ref[...], ws_ref[...], preferred_element_type=jnp.float32)
           + jnp.dot(agg, wn_ref[...], preferred_element_type=jnp.float32)
           + b_ref[...])
    out_ref[...] = jnp.maximum(out, 0.0)


def _row_spec(w):
    return pl.BlockSpec((BN, w), lambda i: (i, 0))


_W_SPEC = pl.BlockSpec((D, D), lambda i: (0, 0))
_B_SPEC = pl.BlockSpec((1, D), lambda i: (0, 0))


def _tc_layer1(x, acc, W_self, W_neigh, b):
    return pl.pallas_call(
        _tc_layer1_kernel,
        grid=(NPAD // BN,),
        in_specs=[_row_spec(D), _row_spec(DEXT), _row_spec(DEXT),
                  _W_SPEC, _W_SPEC, _B_SPEC],
        out_specs=_row_spec(D),
        out_shape=jax.ShapeDtypeStruct((NPAD, D), jnp.float32),
    )(x, acc[0], acc[1], W_self, W_neigh, b.reshape(1, D))


def _tc_layer2(h, acc, deg0, deg1, W_self, W_neigh, b):
    return pl.pallas_call(
        _tc_layer2_kernel,
        grid=(NPAD // BN,),
        in_specs=[_row_spec(D), _row_spec(D), _row_spec(D),
                  _row_spec(1), _row_spec(1), _W_SPEC, _W_SPEC, _B_SPEC],
        out_specs=_row_spec(D),
        out_shape=jax.ShapeDtypeStruct((NPAD, D), jnp.float32),
    )(h, acc[0], acc[1], deg0, deg1, W_self, W_neigh, b.reshape(1, D))


def kernel(x, edge_index, W_self1, W_neigh1, b1, W_self2, W_neigh2, b2):
    src = edge_index[0]
    dst = edge_index[1]
    pad_e = EPAD - E
    src_t = jnp.concatenate(
        [src, jnp.zeros((pad_e,), jnp.int32)]).reshape(NW * KCH, CHUNK)
    dst_t = jnp.concatenate(
        [dst, jnp.full((pad_e,), DUMMY, jnp.int32)]).reshape(NW * KCH, CHUNK)

    xe = jnp.zeros((NPAD, DEXT), jnp.float32)
    xe = xe.at[:N, :D].set(x).at[:N, D].set(1.0)

    acc1 = _sc_aggregate(xe, src_t, dst_t,
                         jnp.zeros((RPT, DEXT), jnp.float32), DEXT)
    h1 = _tc_layer1(xe[:, :D], acc1, W_self1, W_neigh1, b1)

    acc2 = _sc_aggregate(h1, src_t, dst_t,
                         jnp.zeros((RPT, D), jnp.float32), D)
    deg0 = acc1[0, :, D:D + 1]
    deg1 = acc1[1, :, D:D + 1]
    h2 = _tc_layer2(h1, acc2, deg0, deg1, W_self2, W_neigh2, b2)
    return h2[:N]


# speedup vs baseline: 3.8541x; 1.1680x over previous
import functools

import jax
import jax.numpy as jnp
from jax import lax
from jax.experimental import pallas as pl
from jax.experimental.pallas import tpu as pltpu
from jax.experimental.pallas import tpu_sc as plsc

N = 10000
E = 320000
D = 128
DEXT = 144

NC = 2
NS = 16
NW = NC * NS

CHUNK = 128
GRP = 16
KCH = 80
EPT = KCH * CHUNK
EPAD = EPT * NW

NTAB = N + 16
RPT = N // NS

BN = 2000


def _sc_aggregate(feat, src_t, dst_t, z_feat, fw):
    scratch = dict(
        src_v=pltpu.VMEM((GRP, CHUNK), jnp.int32),
        dst_v=pltpu.VMEM((GRP, CHUNK), jnp.int32),
        rows0_v=pltpu.VMEM((CHUNK, fw), jnp.float32),
        rows1_v=pltpu.VMEM((CHUNK, fw), jnp.float32),
        sg0=pltpu.SemaphoreType.DMA,
        sg1=pltpu.SemaphoreType.DMA,
        ss0=pltpu.SemaphoreType.DMA,
        ss1=pltpu.SemaphoreType.DMA,
        acc_sh=pltpu.VMEM_SHARED((N, fw), jnp.float32),
    )

    mesh = plsc.VectorSubcoreMesh(core_axis_name="c", subcore_axis_name="s")

    @functools.partial(
        pl.kernel,
        out_type=jax.ShapeDtypeStruct((NC, N, fw), jnp.float32),
        mesh=mesh, scratch_types=scratch,
        compiler_params=pltpu.CompilerParams(use_tc_tiling_on_sc=False))
    def run(feat_hbm, src_hbm, dst_hbm, zf_hbm, acc_out, *,
            src_v, dst_v, rows0_v, rows1_v, sg0, sg1, ss0, ss1, acc_sh):
        c = lax.axis_index("c")
        s = lax.axis_index("s")
        wid = s * NC + c
        base = s * RPT
        bufs = (rows0_v, rows1_v)
        sgs = (sg0, sg1)
        sss = (ss0, ss1)

        pltpu.sync_copy(zf_hbm, acc_sh.at[pl.ds(base, RPT)])
        plsc.subcore_barrier()

        def body(g, carry):
            off = wid * KCH + g * GRP
            pltpu.sync_copy(src_hbm.at[pl.ds(off, GRP)], src_v)
            pltpu.sync_copy(dst_hbm.at[pl.ds(off, GRP)], dst_v)
            gcp = pltpu.async_copy(feat_hbm.at[src_v.at[0]], bufs[0], sgs[0])
            scp = [None, None]
            for j in range(GRP):
                p = j % 2
                if j + 1 < GRP:
                    if scp[1 - p] is not None:
                        scp[1 - p].wait()
                    ncp = pltpu.async_copy(
                        feat_hbm.at[src_v.at[j + 1]], bufs[1 - p], sgs[1 - p])
                gcp.wait()
                scp[p] = pltpu.async_copy(
                    bufs[p], acc_sh.at[dst_v.at[j]], sss[p], add=True)
                if j + 1 < GRP:
                    gcp = ncp
            scp[0].wait()
            scp[1].wait()
            return carry

        lax.fori_loop(0, KCH // GRP, body, 0)
        plsc.subcore_barrier()

        pltpu.sync_copy(acc_sh.at[pl.ds(base, RPT)],
                        acc_out.at[c].at[pl.ds(base, RPT)])

    return run(feat, src_t, dst_t, z_feat)


def _tc_layer1_kernel(x_ref, a0_ref, a1_ref, ws_ref, wn_ref, b_ref, out_ref):
    acc = a0_ref[...] + a1_ref[...]
    deg = acc[:, D:D + 1]
    inv = 1.0 / jnp.maximum(deg, 1.0)
    agg = acc[:, :D] * inv
    out = (jnp.dot(x_ref[...], ws_ref[...], preferred_element_type=jnp.float32)
           + jnp.dot(agg, wn_ref[...], preferred_element_type=jnp.float32)
           + b_ref[...])
    out_ref[...] = jnp.maximum(out, 0.0)


def _tc_layer2_kernel(h_ref, a0_ref, a1_ref, d0_ref, d1_ref,
                      ws_ref, wn_ref, b_ref, out_ref):
    deg = d0_ref[...] + d1_ref[...]
    inv = 1.0 / jnp.maximum(deg, 1.0)
    agg = (a0_ref[...] + a1_ref[...]) * inv
    out = (jnp.dot(h_ref[...], ws_ref[...], preferred_element_type=jnp.float32)
           + jnp.dot(agg, wn_ref[...], preferred_element_type=jnp.float32)
           + b_ref[...])
    out_ref[...] = jnp.maximum(out, 0.0)


def _row_spec(w):
    return pl.BlockSpec((BN, w), lambda i: (i, 0))


_W_SPEC = pl.BlockSpec((D, D), lambda i: (0, 0))
_B_SPEC = pl.BlockSpec((1, D), lambda i: (0, 0))


def _tc_layer1(x, acc, W_self, W_neigh, b):
    return pl.pallas_call(
        _tc_layer1_kernel,
        grid=(N // BN,),
        in_specs=[_row_spec(D), _row_spec(DEXT), _row_spec(DEXT),
                  _W_SPEC, _W_SPEC, _B_SPEC],
        out_specs=_row_spec(D),
        out_shape=jax.ShapeDtypeStruct((N, D), jnp.float32),
    )(x, acc[0], acc[1], W_self, W_neigh, b.reshape(1, D))


def _tc_layer2(h, acc, deg0, deg1, W_self, W_neigh, b):
    return pl.pallas_call(
        _tc_layer2_kernel,
        grid=(N // BN,),
        in_specs=[_row_spec(D), _row_spec(D), _row_spec(D),
                  _row_spec(1), _row_spec(1), _W_SPEC, _W_SPEC, _B_SPEC],
        out_specs=_row_spec(D),
        out_shape=jax.ShapeDtypeStruct((N, D), jnp.float32),
    )(h, acc[0], acc[1], deg0, deg1, W_self, W_neigh, b.reshape(1, D))


def kernel(x, edge_index, W_self1, W_neigh1, b1, W_self2, W_neigh2, b2):
    src = edge_index[0]
    dst = edge_index[1]
    pad_e = EPAD - E
    src_t = jnp.concatenate(
        [src, jnp.full((pad_e,), N, jnp.int32)]).reshape(NW * KCH, CHUNK)
    dst_t = jnp.concatenate(
        [dst, jnp.zeros((pad_e,), jnp.int32)]).reshape(NW * KCH, CHUNK)

    xe = jnp.zeros((NTAB, DEXT), jnp.float32)
    xe = xe.at[:N, :D].set(x).at[:N, D].set(1.0)

    acc1 = _sc_aggregate(xe, src_t, dst_t,
                         jnp.zeros((RPT, DEXT), jnp.float32), DEXT)
    h1 = _tc_layer1(x, acc1, W_self1, W_neigh1, b1)

    h1p = jnp.zeros((NTAB, D), jnp.float32).at[:N].set(h1)
    acc2 = _sc_aggregate(h1p, src_t, dst_t,
                         jnp.zeros((RPT, D), jnp.float32), D)
    deg0 = acc1[0, :, D:D + 1]
    deg1 = acc1[1, :, D:D + 1]
    return _tc_layer2(h1, acc2, deg0, deg1, W_self2, W_neigh2, b2)
